# baseline (speedup 1.0000x reference)
import functools
import jax
import jax.numpy as jnp
from jax import lax
from jax.experimental import pallas as pl
from jax.experimental.pallas import tpu as pltpu
from jax.experimental.pallas import tpu_sc as plsc

D_MODEL = 1024
NC, NS = 2, 16
NW = NC * NS
CH = 8
NBUF = 3


def _sc_body(x_hbm, p_hbm, o_hbm, buf, sem_in0, sem_in1, sem_in2,
             sem_out0, sem_out1, sem_out2):
    B = 4
    S = 8192
    rows_per_w = S // NW
    nchunks = rows_per_w // CH
    wid = lax.axis_index("s") * NC + lax.axis_index("c")
    seq0 = wid * rows_per_w
    sems_in = (sem_in0, sem_in1, sem_in2)
    sems_out = (sem_out0, sem_out1, sem_out2)

    def in_copies(c, par):
        base = seq0 + c * CH
        return [
            pltpu.make_async_copy(
                p_hbm.at[pl.ds(base, CH)], buf.at[par, 0], sems_in[par]),
            pltpu.make_async_copy(
                x_hbm.at[:, pl.ds(base, CH)], buf.at[par, pl.ds(1, B)],
                sems_in[par]),
        ]

    def out_copies(c, par):
        base = seq0 + c * CH
        return [pltpu.make_async_copy(
            buf.at[par, pl.ds(1, B)], o_hbm.at[:, pl.ds(base, CH)],
            sems_out[par])]

    def compute(par):
        def jbody(j, carry):
            for u in range(2):
                col = j * 32 + u * 16
                for r in range(CH):
                    p = buf[par, 0, r, pl.ds(col, 16)]
                    for b in range(B):
                        buf[par, 1 + b, r, pl.ds(col, 16)] = (
                            buf[par, 1 + b, r, pl.ds(col, 16)] + p)
            return carry
        lax.fori_loop(0, D_MODEL // 32, jbody, 0)

    def step(c, par, prefetch_c):
        if prefetch_c is not None:
            pre_par = (par + 2) % NBUF

            @pl.when(prefetch_c >= NBUF)
            def _drain():
                for cp in out_copies(prefetch_c - NBUF, pre_par):
                    cp.wait()

            for cp in in_copies(prefetch_c, pre_par):
                cp.start()
        for cp in in_copies(c, par):
            cp.wait()
        compute(par)
        for cp in out_copies(c, par):
            cp.start()

    ngroups = nchunks // NBUF

    for c in range(2):
        for cp in in_copies(c, c % NBUF):
            cp.start()

    def gbody(g, carry):
        c0 = g * NBUF
        for par in range(NBUF):
            c = c0 + par
            step(c, par, c + 2)
        return carry

    lax.fori_loop(0, ngroups - 1, gbody, 0)
    for k in range((ngroups - 1) * NBUF, nchunks):
        step(k, k % NBUF, k + 2 if k + 2 < nchunks else None)
    for k in range(nchunks - NBUF, nchunks):
        for cp in out_copies(k, k % NBUF):
            cp.wait()


def kernel(x, pos_weight):
    B, S, D = x.shape
    mesh = plsc.VectorSubcoreMesh(core_axis_name="c", subcore_axis_name="s")
    run = functools.partial(
        pl.kernel,
        mesh=mesh,
        out_type=jax.ShapeDtypeStruct((B, S, D), jnp.float32),
        scratch_types=[
            pltpu.VMEM((NBUF, 5, CH, D), jnp.float32),
            pltpu.SemaphoreType.DMA,
            pltpu.SemaphoreType.DMA,
            pltpu.SemaphoreType.DMA,
            pltpu.SemaphoreType.DMA,
            pltpu.SemaphoreType.DMA,
            pltpu.SemaphoreType.DMA,
        ],
    )(_sc_body)
    return run(x, pos_weight[:S])


# speedup vs baseline: 2.8427x; 2.8427x over previous
import functools
import jax
import jax.numpy as jnp
from jax import lax
from jax.experimental import pallas as pl
from jax.experimental.pallas import tpu as pltpu
from jax.experimental.pallas import tpu_sc as plsc

D_MODEL = 1024
NC, NS = 2, 16
NW = NC * NS
CH = 8
NBUF = 3


def _sc_body(x_hbm, p_hbm, o_hbm, buf, sem_in0, sem_in1, sem_in2,
             sem_out0, sem_out1, sem_out2):
    B = 4
    S = 8192
    rows_per_w = S // NW
    nchunks = rows_per_w // CH
    wid = lax.axis_index("s") * NC + lax.axis_index("c")
    seq0 = wid * rows_per_w
    sems_in = (sem_in0, sem_in1, sem_in2)
    sems_out = (sem_out0, sem_out1, sem_out2)

    def in_copies(c, par):
        base = seq0 + c * CH
        return [
            pltpu.make_async_copy(
                p_hbm.at[pl.ds(base, CH)], buf.at[par, 0], sems_in[par]),
            pltpu.make_async_copy(
                x_hbm.at[:, pl.ds(base, CH)], buf.at[par, pl.ds(1, B)],
                sems_in[par]),
        ]

    def out_copies(c, par):
        base = seq0 + c * CH
        return [pltpu.make_async_copy(
            buf.at[par, pl.ds(1, B)], o_hbm.at[:, pl.ds(base, CH)],
            sems_out[par])]

    def compute(par):
        def rbody(r, carry):
            def jbody(j, c2):
                col = j * 16
                p = buf[par, 0, r, pl.ds(col, 16)]
                for b in range(B):
                    buf[par, 1 + b, r, pl.ds(col, 16)] = (
                        buf[par, 1 + b, r, pl.ds(col, 16)] + p)
                return c2
            lax.fori_loop(0, D_MODEL // 16, jbody, carry)
            return carry
        lax.fori_loop(0, CH, rbody, 0)

    def step(c, par, prefetch_c):
        if prefetch_c is not None:
            pre_par = (par + 2) % NBUF

            @pl.when(prefetch_c >= NBUF)
            def _drain():
                for cp in out_copies(prefetch_c - NBUF, pre_par):
                    cp.wait()

            for cp in in_copies(prefetch_c, pre_par):
                cp.start()
        for cp in in_copies(c, par):
            cp.wait()
        compute(par)
        for cp in out_copies(c, par):
            cp.start()

    ngroups = nchunks // NBUF

    for c in range(2):
        for cp in in_copies(c, c % NBUF):
            cp.start()

    def gbody(g, carry):
        c0 = g * NBUF
        for par in range(NBUF):
            c = c0 + par
            step(c, par, c + 2)
        return carry

    lax.fori_loop(0, ngroups - 1, gbody, 0)
    for k in range((ngroups - 1) * NBUF, nchunks):
        step(k, k % NBUF, k + 2 if k + 2 < nchunks else None)
    for k in range(nchunks - NBUF, nchunks):
        for cp in out_copies(k, k % NBUF):
            cp.wait()


def kernel(x, pos_weight):
    B, S, D = x.shape
    mesh = plsc.VectorSubcoreMesh(core_axis_name="c", subcore_axis_name="s")
    run = functools.partial(
        pl.kernel,
        mesh=mesh,
        out_type=jax.ShapeDtypeStruct((B, S, D), jnp.float32),
        scratch_types=[
            pltpu.VMEM((NBUF, 5, CH, D), jnp.float32),
            pltpu.SemaphoreType.DMA,
            pltpu.SemaphoreType.DMA,
            pltpu.SemaphoreType.DMA,
            pltpu.SemaphoreType.DMA,
            pltpu.SemaphoreType.DMA,
            pltpu.SemaphoreType.DMA,
        ],
    )(_sc_body)
    return run(x, pos_weight[:S])


# speedup vs baseline: 3.1539x; 1.1095x over previous
import functools
import jax
import jax.numpy as jnp
from jax import lax
from jax.experimental import pallas as pl
from jax.experimental.pallas import tpu as pltpu
from jax.experimental.pallas import tpu_sc as plsc

D_MODEL = 1024
NC, NS = 2, 16
NW = NC * NS
CH = 8
NBUF = 3


def _sc_body(x_hbm, p_hbm, o_hbm, buf, sem_in0, sem_in1, sem_in2,
             sem_out0, sem_out1, sem_out2):
    B = 4
    S = 8192
    rows_per_w = S // NW
    nchunks = rows_per_w // CH
    wid = lax.axis_index("s") * NC + lax.axis_index("c")
    seq0 = wid * rows_per_w
    sems_in = (sem_in0, sem_in1, sem_in2)
    sems_out = (sem_out0, sem_out1, sem_out2)

    def in_copies(c, par):
        base = seq0 + c * CH
        return [
            pltpu.make_async_copy(
                p_hbm.at[pl.ds(base, CH)], buf.at[par, 0], sems_in[par]),
            pltpu.make_async_copy(
                x_hbm.at[:, pl.ds(base, CH)], buf.at[par, pl.ds(1, B)],
                sems_in[par]),
        ]

    def out_copies(c, par):
        base = seq0 + c * CH
        return [pltpu.make_async_copy(
            buf.at[par, pl.ds(1, B)], o_hbm.at[:, pl.ds(base, CH)],
            sems_out[par])]

    def compute(par):
        def jbody(j, carry):
            col = j * 16
            for r in range(CH):
                p = buf[par, 0, r, pl.ds(col, 16)]
                for b in range(B):
                    buf[par, 1 + b, r, pl.ds(col, 16)] = (
                        buf[par, 1 + b, r, pl.ds(col, 16)] + p)
            return carry
        lax.fori_loop(0, D_MODEL // 16, jbody, 0)

    def step(c, par, prefetch_c):
        for cp in in_copies(c, par):
            cp.wait()
        compute(par)
        for cp in out_copies(c, par):
            cp.start()
        if prefetch_c is not None:
            pre_par = (par + 2) % NBUF

            @pl.when(prefetch_c >= NBUF)
            def _drain():
                for cp in out_copies(prefetch_c - NBUF, pre_par):
                    cp.wait()

            for cp in in_copies(prefetch_c, pre_par):
                cp.start()

    ngroups = nchunks // NBUF

    for c in range(2):
        for cp in in_copies(c, c % NBUF):
            cp.start()

    def gbody(g, carry):
        c0 = g * NBUF
        for par in range(NBUF):
            c = c0 + par
            step(c, par, c + 2)
        return carry

    lax.fori_loop(0, ngroups, gbody, 0)
    for k in range(ngroups * NBUF, nchunks):
        step(k, k % NBUF, None)
    for k in range(nchunks - NBUF, nchunks):
        for cp in out_copies(k, k % NBUF):
            cp.wait()


def kernel(x, pos_weight):
    B, S, D = x.shape
    mesh = plsc.VectorSubcoreMesh(core_axis_name="c", subcore_axis_name="s")
    run = functools.partial(
        pl.kernel,
        mesh=mesh,
        out_type=jax.ShapeDtypeStruct((B, S, D), jnp.float32),
        scratch_types=[
            pltpu.VMEM((NBUF, 5, CH, D), jnp.float32),
            pltpu.SemaphoreType.DMA,
            pltpu.SemaphoreType.DMA,
            pltpu.SemaphoreType.DMA,
            pltpu.SemaphoreType.DMA,
            pltpu.SemaphoreType.DMA,
            pltpu.SemaphoreType.DMA,
        ],
    )(_sc_body)
    return run(x, pos_weight[:S])
